# baseline (speedup 1.0000x reference)
import functools

import jax
import jax.numpy as jnp
from jax import lax
from jax.experimental import pallas as pl
from jax.experimental.pallas import tpu as pltpu
from jax.experimental.pallas import tpu_sc as plsc

NUM_BUCKETS = 1000000
EMBED_DIM = 32
ROW = 200
NROWS = 4096
NW = 32
JT = ROW // 8
IT = NROWS // 128
GSZ = 8 * 128

C_PH = 761000
C_PL = 435761
C_CH = 503000
C_CL = 40503


def _hash16(prev, cur):
    ph = lax.div(prev, jnp.int32(1000))
    plo = prev - ph * jnp.int32(1000)
    ch = lax.div(cur, jnp.int32(1000))
    clo = cur - ch * jnp.int32(1000)
    s = (jnp.int32(C_PH) * ph + jnp.int32(C_PL) * plo
         + jnp.int32(C_CH) * ch + jnp.int32(C_CL) * clo)
    return s % jnp.int32(NUM_BUCKETS)


def _make_sc_call():
    mesh = plsc.VectorSubcoreMesh(core_axis_name="c", subcore_axis_name="s")

    @functools.partial(
        pl.kernel,
        mesh=mesh,
        out_type=jax.ShapeDtypeStruct((ROW, 4, IT, 8, 128), jnp.float32),
        scratch_types=[
            pltpu.VMEM((8, 128), jnp.int32),
            pltpu.VMEM((128,), jnp.int32),
            pltpu.VMEM((GSZ,), jnp.int32),
            pltpu.VMEM((GSZ, EMBED_DIM), jnp.float32),
            pltpu.VMEM((8, 4, 8, 128), jnp.float32),
            pltpu.SemaphoreType.DMA,
        ],
        compiler_params=pltpu.CompilerParams(use_tc_tiling_on_sc=False,
                                             needs_layout_passes=False),
    )
    def sc_gather(ids_hbm, table_hbm, out_hbm, ids_v, pcol_v, idx_v, rows_v,
                  out_v, sem):
        w = lax.axis_index("s") * 2 + lax.axis_index("c")

        def group_body(g, carry):
            pltpu.sync_copy(ids_hbm.at[g, w], ids_v)

            @pl.when(g > 0)
            def _():
                pltpu.sync_copy(ids_hbm.at[g - 1, w, jnp.int32(7)], pcol_v)

            gpos = jnp.full((16,), g, jnp.int32)

            def hash_body(k, c):
                k16 = k * jnp.int32(16)
                for j in range(8):
                    cur = ids_v[jnp.int32(j), pl.ds(k16, 16)]
                    if j == 0:
                        prev = jnp.where(gpos > 0, pcol_v[pl.ds(k16, 16)], 0)
                    else:
                        prev = ids_v[jnp.int32(j - 1), pl.ds(k16, 16)]
                    idx_v[pl.ds(jnp.int32(j * 128) + k16, 16)] = \
                        _hash16(prev, cur)
                return c

            lax.fori_loop(jnp.int32(0), jnp.int32(8), hash_body, jnp.int32(0))

            copies = [
                pltpu.async_copy(
                    table_hbm.at[idx_v.at[pl.ds(j * 128, 128)]],
                    rows_v.at[pl.ds(j * 128, 128)],
                    sem)
                for j in range(8)
            ]
            for c in copies:
                c.wait()

            iota = lax.iota(jnp.int32, 16)

            def tpose_body(j, c):
                j128 = j * jnp.int32(128)
                for e_t in range(4):
                    for ee in range(8):
                        col = jnp.full((16,), e_t * 8 + ee, jnp.int32)
                        for kk in range(8):
                            rr = j128 + jnp.int32(kk * 16) + iota
                            out_v[j, jnp.int32(e_t), jnp.int32(ee),
                                  pl.ds(kk * 16, 16)] = \
                                plsc.load_gather(rows_v, [rr, col])
                for e_t in range(4):
                    pltpu.sync_copy(
                        out_v.at[j, jnp.int32(e_t)],
                        out_hbm.at[g * jnp.int32(8) + j, jnp.int32(e_t), w])
                return c

            lax.fori_loop(jnp.int32(0), jnp.int32(8), tpose_body, jnp.int32(0))
            return carry

        lax.fori_loop(jnp.int32(0), jnp.int32(JT), group_body, jnp.int32(0))

    return sc_gather


_SC_GATHER = _make_sc_call()


def kernel(input_ids, table):
    ids32 = input_ids.astype(jnp.int32)
    ids_x = ids32.reshape(IT, 128, JT, 8).transpose(2, 0, 3, 1)
    out5 = _SC_GATHER(ids_x, table)
    return out5.transpose(2, 4, 0, 1, 3).reshape(NROWS, ROW, EMBED_DIM)


# speedup vs baseline: 1.0587x; 1.0587x over previous
import functools

import jax
import jax.numpy as jnp
from jax import lax
from jax.experimental import pallas as pl
from jax.experimental.pallas import tpu as pltpu
from jax.experimental.pallas import tpu_sc as plsc

NUM_BUCKETS = 1000000
EMBED_DIM = 32
ROW = 200
NROWS = 4096
NW = 32
JT = ROW // 8
IT = NROWS // 128
GSZ = 8 * 128

C_PH = 761000
C_PL = 435761
C_CH = 503000
C_CL = 40503


def _hash16(prev, cur):
    ph = lax.div(prev, jnp.int32(1000))
    plo = prev - ph * jnp.int32(1000)
    ch = lax.div(cur, jnp.int32(1000))
    clo = cur - ch * jnp.int32(1000)
    s = (jnp.int32(C_PH) * ph + jnp.int32(C_PL) * plo
         + jnp.int32(C_CH) * ch + jnp.int32(C_CL) * clo)
    return s % jnp.int32(NUM_BUCKETS)


def _make_sc_call():
    mesh = plsc.VectorSubcoreMesh(core_axis_name="c", subcore_axis_name="s")

    @functools.partial(
        pl.kernel,
        mesh=mesh,
        out_type=jax.ShapeDtypeStruct((ROW, 4, IT, 8, 128), jnp.float32),
        scratch_types=[
            pltpu.VMEM((8, 128), jnp.int32),
            pltpu.VMEM((128,), jnp.int32),
            pltpu.VMEM((GSZ,), jnp.int32),
            pltpu.VMEM((GSZ, EMBED_DIM), jnp.float32),
            pltpu.VMEM((32, 8, 128), jnp.float32),
            pltpu.SemaphoreType.DMA,
            pltpu.SemaphoreType.DMA,
            pltpu.SemaphoreType.DMA,
        ],
        compiler_params=pltpu.CompilerParams(use_tc_tiling_on_sc=False,
                                             needs_layout_passes=False),
    )
    def sc_gather(ids_hbm, table_hbm, out_hbm, ids_v, pcol_v, idx_v, rows_v,
                  out_v, sem, sem_ids, sem_out):
        w = lax.axis_index("s") * 2 + lax.axis_index("c")

        def group_body(g, carry):
            in_copies = [pltpu.async_copy(ids_hbm.at[g, w], ids_v, sem_ids)]

            @pl.when(g > 0)
            def _():
                pltpu.async_copy(ids_hbm.at[g - 1, w, jnp.int32(7)], pcol_v,
                                 sem_ids).wait()

            in_copies[0].wait()

            gpos = jnp.full((16,), g, jnp.int32)

            def hash_body(k, c):
                k16 = k * jnp.int32(16)
                for j in range(8):
                    cur = ids_v[jnp.int32(j), pl.ds(k16, 16)]
                    if j == 0:
                        prev = jnp.where(gpos > 0, pcol_v[pl.ds(k16, 16)], 0)
                    else:
                        prev = ids_v[jnp.int32(j - 1), pl.ds(k16, 16)]
                    idx_v[pl.ds(jnp.int32(j * 128) + k16, 16)] = \
                        _hash16(prev, cur)
                return c

            lax.fori_loop(jnp.int32(0), jnp.int32(8), hash_body, jnp.int32(0))

            copies = [
                pltpu.async_copy(
                    table_hbm.at[idx_v.at[pl.ds(j * 128, 128)]],
                    rows_v.at[pl.ds(j * 128, 128)],
                    sem)
                for j in range(8)
            ]
            for c in copies:
                c.wait()

            iota = lax.iota(jnp.int32, 16)

            def tpose_body(j, c):
                j128 = j * jnp.int32(128)
                j4 = j * jnp.int32(4)
                for e_t in range(4):
                    for ee in range(8):
                        col = jnp.full((16,), e_t * 8 + ee, jnp.int32)
                        for kk in range(8):
                            rr = j128 + jnp.int32(kk * 16) + iota
                            out_v[j4 + jnp.int32(e_t), jnp.int32(ee),
                                  pl.ds(kk * 16, 16)] = \
                                plsc.load_gather(rows_v, [rr, col])
                for e_t in range(4):
                    pltpu.async_copy(
                        out_v.at[j4 + jnp.int32(e_t)],
                        out_hbm.at[g * jnp.int32(8) + j, jnp.int32(e_t), w],
                        sem_out)
                return c

            lax.fori_loop(jnp.int32(0), jnp.int32(8), tpose_body, jnp.int32(0))
            pltpu.make_async_copy(out_hbm.at[jnp.int32(0), jnp.int32(0)],
                                  out_v, sem_out).wait()
            return carry

        lax.fori_loop(jnp.int32(0), jnp.int32(JT), group_body, jnp.int32(0))

    return sc_gather


_SC_GATHER = _make_sc_call()


def kernel(input_ids, table):
    ids32 = input_ids.astype(jnp.int32)
    ids_x = ids32.reshape(IT, 128, JT, 8).transpose(2, 0, 3, 1)
    out5 = _SC_GATHER(ids_x, table)
    return out5.transpose(2, 4, 0, 1, 3).reshape(NROWS, ROW, EMBED_DIM)


# speedup vs baseline: 1.1315x; 1.0688x over previous
import functools

import jax
import jax.numpy as jnp
from jax import lax
from jax.experimental import pallas as pl
from jax.experimental.pallas import tpu as pltpu
from jax.experimental.pallas import tpu_sc as plsc

NUM_BUCKETS = 1000000
EMBED_DIM = 32
ROW = 200
NROWS = 4096
NW = 32
JT = ROW // 8
IT = NROWS // 128
GSZ = 8 * 128
PITCH = EMBED_DIM + 1

C_PH = 761000
C_PL = 435761
C_CH = 503000
C_CL = 40503


def _hash16(prev, cur):
    ph = lax.div(prev, jnp.int32(1000))
    plo = prev - ph * jnp.int32(1000)
    ch = lax.div(cur, jnp.int32(1000))
    clo = cur - ch * jnp.int32(1000)
    s = (jnp.int32(C_PH) * ph + jnp.int32(C_PL) * plo
         + jnp.int32(C_CH) * ch + jnp.int32(C_CL) * clo)
    return s % jnp.int32(NUM_BUCKETS)


def _make_sc_call():
    mesh = plsc.VectorSubcoreMesh(core_axis_name="c", subcore_axis_name="s")

    @functools.partial(
        pl.kernel,
        mesh=mesh,
        out_type=jax.ShapeDtypeStruct((ROW, 4, IT, 8, 128), jnp.float32),
        scratch_types=[
            pltpu.VMEM((2, 8, 128), jnp.int32),
            pltpu.VMEM((2, 128), jnp.int32),
            pltpu.VMEM((2, GSZ), jnp.int32),
            pltpu.VMEM((GSZ, EMBED_DIM), jnp.float32),
            pltpu.VMEM((GSZ, PITCH), jnp.float32),
            pltpu.VMEM((32, 8, 128), jnp.float32),
            pltpu.SemaphoreType.DMA,
            pltpu.SemaphoreType.DMA,
            pltpu.SemaphoreType.DMA,
        ],
        compiler_params=pltpu.CompilerParams(use_tc_tiling_on_sc=False,
                                             needs_layout_passes=False),
    )
    def sc_gather(ids_hbm, table_hbm, out_hbm, ids_v, pcol_v, idx_v, rows_v,
                  st_v, out_v, sem_ids, sem_g, sem_out):
        w = lax.axis_index("s") * 2 + lax.axis_index("c")
        iota = lax.iota(jnp.int32, 16)

        def fire_in(g, P, with_pcol=True):
            pltpu.async_copy(ids_hbm.at[g, w], ids_v.at[P], sem_ids)
            if with_pcol:
                pltpu.async_copy(ids_hbm.at[g - 1, w, jnp.int32(7)],
                                 pcol_v.at[P], sem_ids)

        def wait_in(P, with_pcol=True):
            pltpu.make_async_copy(ids_hbm.at[jnp.int32(0), w],
                                  ids_v.at[P], sem_ids).wait()
            if with_pcol:
                pltpu.make_async_copy(
                    ids_hbm.at[jnp.int32(0), w, jnp.int32(0)],
                    pcol_v.at[P], sem_ids).wait()

        def hash_group(g, P):
            gpos = jnp.full((16,), g, jnp.int32)

            def hash_body(k, c):
                k16 = k * jnp.int32(16)
                for j in range(8):
                    cur = ids_v[P, jnp.int32(j), pl.ds(k16, 16)]
                    if j == 0:
                        prev = jnp.where(gpos > 0,
                                         pcol_v[P, pl.ds(k16, 16)], 0)
                    else:
                        prev = ids_v[P, jnp.int32(j - 1), pl.ds(k16, 16)]
                    idx_v[P, pl.ds(jnp.int32(j * 128) + k16, 16)] = \
                        _hash16(prev, cur)
                return c

            lax.fori_loop(jnp.int32(0), jnp.int32(8), hash_body,
                          jnp.int32(0))

        def fire_gathers(P):
            for j in range(8):
                pltpu.async_copy(
                    table_hbm.at[idx_v.at[P, pl.ds(j * 128, 128)]],
                    rows_v.at[pl.ds(j * 128, 128)],
                    sem_g)

        def drain_gathers():
            pltpu.make_async_copy(table_hbm.at[pl.ds(jnp.int32(0), GSZ)],
                                  rows_v, sem_g).wait()

        def drain_out():
            pltpu.make_async_copy(out_hbm.at[jnp.int32(0), jnp.int32(0)],
                                  out_v, sem_out).wait()

        def stage_rows():
            def stage_body(m, c):
                m8 = m * jnp.int32(8)
                for r in range(8):
                    row = m8 + jnp.int32(r)
                    a = rows_v[row, pl.ds(jnp.int32(0), 16)]
                    b = rows_v[row, pl.ds(jnp.int32(16), 16)]
                    st_v[row, pl.ds(jnp.int32(0), 16)] = a
                    st_v[row, pl.ds(jnp.int32(16), 16)] = b
                return c

            lax.fori_loop(jnp.int32(0), jnp.int32(GSZ // 8), stage_body,
                          jnp.int32(0))

        def transpose_and_out(g):
            def tpose_body(j, c):
                j128 = j * jnp.int32(128)
                j4 = j * jnp.int32(4)
                for kk in range(8):
                    rr = j128 + jnp.int32(kk * 16) + iota
                    for e_t in range(4):
                        for ee in range(8):
                            col = jnp.full((16,), e_t * 8 + ee, jnp.int32)
                            out_v[j4 + jnp.int32(e_t), jnp.int32(ee),
                                  pl.ds(kk * 16, 16)] = \
                                plsc.load_gather(st_v, [rr, col])
                for e_t in range(4):
                    pltpu.async_copy(
                        out_v.at[j4 + jnp.int32(e_t)],
                        out_hbm.at[g * jnp.int32(8) + j, jnp.int32(e_t), w],
                        sem_out)
                return c

            lax.fori_loop(jnp.int32(0), jnp.int32(8), tpose_body,
                          jnp.int32(0))

        fire_in(jnp.int32(0), jnp.int32(0), with_pcol=False)
        wait_in(jnp.int32(0), with_pcol=False)
        hash_group(jnp.int32(0), jnp.int32(0))
        fire_gathers(jnp.int32(0))
        fire_in(jnp.int32(1), jnp.int32(1))

        def group_body(g, carry):
            P = g & jnp.int32(1)
            Q = jnp.int32(1) - P

            @pl.when(g < JT - 1)
            def _():
                wait_in(Q)
                hash_group(g + jnp.int32(1), Q)

            drain_gathers()
            stage_rows()

            @pl.when(g < JT - 1)
            def _():
                fire_gathers(Q)

            @pl.when(g + 2 <= JT - 1)
            def _():
                fire_in(g + jnp.int32(2), P)

            @pl.when(g > 0)
            def _():
                drain_out()

            transpose_and_out(g)
            return carry

        lax.fori_loop(jnp.int32(0), jnp.int32(JT), group_body, jnp.int32(0))
        drain_out()

    return sc_gather


_SC_GATHER = _make_sc_call()


def kernel(input_ids, table):
    ids32 = input_ids.astype(jnp.int32)
    ids_x = ids32.reshape(IT, 128, JT, 8).transpose(2, 0, 3, 1)
    out5 = _SC_GATHER(ids_x, table)
    return out5.transpose(2, 4, 0, 1, 3).reshape(NROWS, ROW, EMBED_DIM)


# speedup vs baseline: 1.1345x; 1.0027x over previous
import functools

import jax
import jax.numpy as jnp
from jax import lax
from jax.experimental import pallas as pl
from jax.experimental.pallas import tpu as pltpu
from jax.experimental.pallas import tpu_sc as plsc

NUM_BUCKETS = 1000000
EMBED_DIM = 32
ROW = 200
NROWS = 4096
NW = 32
JT = ROW // 8
IT = NROWS // 128
GSZ = 8 * 128
PITCH = EMBED_DIM + 1

C_PH = 761000
C_PL = 435761
C_CH = 503000
C_CL = 40503


def _hash16(prev, cur):
    ph = lax.div(prev, jnp.int32(1000))
    plo = prev - ph * jnp.int32(1000)
    ch = lax.div(cur, jnp.int32(1000))
    clo = cur - ch * jnp.int32(1000)
    s = (jnp.int32(C_PH) * ph + jnp.int32(C_PL) * plo
         + jnp.int32(C_CH) * ch + jnp.int32(C_CL) * clo)
    return s % jnp.int32(NUM_BUCKETS)


def _make_sc_call():
    mesh = plsc.VectorSubcoreMesh(core_axis_name="c", subcore_axis_name="s")

    @functools.partial(
        pl.kernel,
        mesh=mesh,
        out_type=jax.ShapeDtypeStruct((ROW, 4, IT, 8, 128), jnp.float32),
        scratch_types=[
            pltpu.VMEM((2, 8, 128), jnp.int32),
            pltpu.VMEM((2, GSZ), jnp.int32),
            pltpu.VMEM((GSZ, EMBED_DIM), jnp.float32),
            pltpu.VMEM((GSZ, PITCH), jnp.float32),
            pltpu.VMEM((8, 4, 8, 128), jnp.float32),
            pltpu.SemaphoreType.DMA,
            pltpu.SemaphoreType.DMA,
            pltpu.SemaphoreType.DMA,
        ],
        compiler_params=pltpu.CompilerParams(use_tc_tiling_on_sc=False,
                                             needs_layout_passes=False),
    )
    def sc_gather(ids_hbm, table_hbm, out_hbm, ids_v, idx_v, rows_v,
                  st_v, out_v, sem_ids, sem_g, sem_out):
        w = lax.axis_index("s") * 2 + lax.axis_index("c")
        iota = lax.iota(jnp.int32, 16)

        def fire_in(g, P):
            pltpu.async_copy(ids_hbm.at[g, w], ids_v.at[P], sem_ids)

        def wait_in(P):
            pltpu.make_async_copy(ids_hbm.at[jnp.int32(0), w],
                                  ids_v.at[P], sem_ids).wait()

        def hash_group(g, P):
            gpos = jnp.full((16,), g, jnp.int32)

            def hash_body(k, c):
                k16 = k * jnp.int32(16)
                for j in range(8):
                    cur = ids_v[P, jnp.int32(j), pl.ds(k16, 16)]
                    if j == 0:
                        prev = jnp.where(
                            gpos > 0,
                            ids_v[jnp.int32(1) - P, jnp.int32(7),
                                  pl.ds(k16, 16)], 0)
                    else:
                        prev = ids_v[P, jnp.int32(j - 1), pl.ds(k16, 16)]
                    idx_v[P, pl.ds(jnp.int32(j * 128) + k16, 16)] = \
                        _hash16(prev, cur)
                return c

            lax.fori_loop(jnp.int32(0), jnp.int32(8), hash_body,
                          jnp.int32(0))

        def fire_gathers(P):
            for j in range(8):
                pltpu.async_copy(
                    table_hbm.at[idx_v.at[P, pl.ds(j * 128, 128)]],
                    rows_v.at[pl.ds(j * 128, 128)],
                    sem_g)

        def drain_gathers():
            pltpu.make_async_copy(table_hbm.at[pl.ds(jnp.int32(0), GSZ)],
                                  rows_v, sem_g).wait()

        def drain_out():
            pltpu.make_async_copy(
                out_hbm.at[pl.ds(jnp.int32(0), 8), :, w],
                out_v, sem_out).wait()

        def stage_rows():
            def stage_body(m, c):
                m8 = m * jnp.int32(8)
                for r in range(8):
                    row = m8 + jnp.int32(r)
                    a = rows_v[row, pl.ds(jnp.int32(0), 16)]
                    b = rows_v[row, pl.ds(jnp.int32(16), 16)]
                    st_v[row, pl.ds(jnp.int32(0), 16)] = a
                    st_v[row, pl.ds(jnp.int32(16), 16)] = b
                return c

            lax.fori_loop(jnp.int32(0), jnp.int32(GSZ // 8), stage_body,
                          jnp.int32(0))

        def transpose_and_out(g):
            def tpose_body(j, c):
                j128 = j * jnp.int32(128)
                for kk in range(8):
                    rr = j128 + jnp.int32(kk * 16) + iota
                    for e_t in range(4):
                        for ee in range(8):
                            col = jnp.full((16,), e_t * 8 + ee, jnp.int32)
                            out_v[j, jnp.int32(e_t), jnp.int32(ee),
                                  pl.ds(kk * 16, 16)] = \
                                plsc.load_gather(st_v, [rr, col])
                return c

            lax.fori_loop(jnp.int32(0), jnp.int32(8), tpose_body,
                          jnp.int32(0))
            pltpu.async_copy(out_v,
                             out_hbm.at[pl.ds(g * jnp.int32(8), 8), :, w],
                             sem_out)

        fire_in(jnp.int32(0), jnp.int32(0))
        wait_in(jnp.int32(0))
        hash_group(jnp.int32(0), jnp.int32(0))
        fire_gathers(jnp.int32(0))
        fire_in(jnp.int32(1), jnp.int32(1))

        def group_body(g, carry):
            P = g & jnp.int32(1)
            Q = jnp.int32(1) - P

            @pl.when(g < JT - 1)
            def _():
                wait_in(Q)
                hash_group(g + jnp.int32(1), Q)

            drain_gathers()
            stage_rows()

            @pl.when(g < JT - 1)
            def _():
                fire_gathers(Q)

            @pl.when(g + 2 <= JT - 1)
            def _():
                fire_in(g + jnp.int32(2), P)

            @pl.when(g > 0)
            def _():
                drain_out()

            transpose_and_out(g)
            return carry

        lax.fori_loop(jnp.int32(0), jnp.int32(JT), group_body, jnp.int32(0))
        drain_out()

    return sc_gather


_SC_GATHER = _make_sc_call()


def kernel(input_ids, table):
    ids32 = input_ids.astype(jnp.int32)
    ids_x = ids32.reshape(IT, 128, JT, 8).transpose(2, 0, 3, 1)
    out5 = _SC_GATHER(ids_x, table)
    return out5.transpose(2, 4, 0, 1, 3).reshape(NROWS, ROW, EMBED_DIM)


# speedup vs baseline: 1.6148x; 1.4233x over previous
import functools

import jax
import jax.numpy as jnp
from jax import lax
from jax.experimental import pallas as pl
from jax.experimental.pallas import tpu as pltpu
from jax.experimental.pallas import tpu_sc as plsc

NUM_BUCKETS = 1000000
EMBED_DIM = 32
ROW = 200
NROWS = 4096
NW = 32
JT = ROW // 8
IT = NROWS // 128
GSZ = 8 * 128
PITCH = EMBED_DIM + 1

C_PH = 219264
C_PL = 435761
C_CH = 475072
C_CL = 40503


def _hash16(prev, cur):
    ph = prev >> jnp.int32(10)
    plo = prev & jnp.int32(1023)
    ch = cur >> jnp.int32(10)
    clo = cur & jnp.int32(1023)
    s = (jnp.int32(C_PH) * ph + jnp.int32(C_PL) * plo
         + jnp.int32(C_CH) * ch + jnp.int32(C_CL) * clo)
    q = (s.astype(jnp.float32) * jnp.float32(1e-6)).astype(jnp.int32)
    r = s - q * jnp.int32(NUM_BUCKETS)
    r = jnp.where(r < 0, r + jnp.int32(NUM_BUCKETS), r)
    r = jnp.where(r >= jnp.int32(NUM_BUCKETS),
                  r - jnp.int32(NUM_BUCKETS), r)
    return r


def _make_sc_call():
    mesh = plsc.VectorSubcoreMesh(core_axis_name="c", subcore_axis_name="s")

    @functools.partial(
        pl.kernel,
        mesh=mesh,
        out_type=jax.ShapeDtypeStruct((ROW, 4, IT, 8, 128), jnp.float32),
        scratch_types=[
            pltpu.VMEM((2, 8, 128), jnp.int32),
            pltpu.VMEM((2, GSZ), jnp.int32),
            pltpu.VMEM((GSZ, EMBED_DIM), jnp.float32),
            pltpu.VMEM((GSZ, PITCH), jnp.float32),
            pltpu.VMEM((8, 4, 8, 128), jnp.float32),
            pltpu.SemaphoreType.DMA,
            pltpu.SemaphoreType.DMA,
            pltpu.SemaphoreType.DMA,
        ],
        compiler_params=pltpu.CompilerParams(use_tc_tiling_on_sc=False,
                                             needs_layout_passes=False),
    )
    def sc_gather(ids_hbm, table_hbm, out_hbm, ids_v, idx_v, rows_v,
                  st_v, out_v, sem_ids, sem_g, sem_out):
        w = lax.axis_index("s") * 2 + lax.axis_index("c")
        iota = lax.iota(jnp.int32, 16)

        def fire_in(g, P):
            pltpu.async_copy(ids_hbm.at[g, w], ids_v.at[P], sem_ids)

        def wait_in(P):
            pltpu.make_async_copy(ids_hbm.at[jnp.int32(0), w],
                                  ids_v.at[P], sem_ids).wait()

        def hash_group(g, P):
            gpos = jnp.full((16,), g, jnp.int32)

            def hash_body(k, c):
                k16 = k * jnp.int32(16)
                for j in range(8):
                    cur = ids_v[P, jnp.int32(j), pl.ds(k16, 16)]
                    if j == 0:
                        prev = jnp.where(
                            gpos > 0,
                            ids_v[jnp.int32(1) - P, jnp.int32(7),
                                  pl.ds(k16, 16)], 0)
                    else:
                        prev = ids_v[P, jnp.int32(j - 1), pl.ds(k16, 16)]
                    idx_v[P, pl.ds(jnp.int32(j * 128) + k16, 16)] = \
                        _hash16(prev, cur)
                return c

            lax.fori_loop(jnp.int32(0), jnp.int32(8), hash_body,
                          jnp.int32(0))

        def fire_gathers(P):
            for j in range(8):
                pltpu.async_copy(
                    table_hbm.at[idx_v.at[P, pl.ds(j * 128, 128)]],
                    rows_v.at[pl.ds(j * 128, 128)],
                    sem_g)

        def drain_gathers():
            pltpu.make_async_copy(table_hbm.at[pl.ds(jnp.int32(0), GSZ)],
                                  rows_v, sem_g).wait()

        def drain_out():
            pltpu.make_async_copy(
                out_hbm.at[pl.ds(jnp.int32(0), 8), :, w],
                out_v, sem_out).wait()

        def stage_rows():
            def stage_body(m, c):
                m8 = m * jnp.int32(8)
                for r in range(8):
                    row = m8 + jnp.int32(r)
                    a = rows_v[row, pl.ds(jnp.int32(0), 16)]
                    b = rows_v[row, pl.ds(jnp.int32(16), 16)]
                    st_v[row, pl.ds(jnp.int32(0), 16)] = a
                    st_v[row, pl.ds(jnp.int32(16), 16)] = b
                return c

            lax.fori_loop(jnp.int32(0), jnp.int32(GSZ // 8), stage_body,
                          jnp.int32(0))

        def transpose_and_out(g):
            def tpose_body(j, c):
                j128 = j * jnp.int32(128)
                for kk in range(8):
                    rr = j128 + jnp.int32(kk * 16) + iota
                    for e_t in range(4):
                        for ee in range(8):
                            col = jnp.full((16,), e_t * 8 + ee, jnp.int32)
                            out_v[j, jnp.int32(e_t), jnp.int32(ee),
                                  pl.ds(kk * 16, 16)] = \
                                plsc.load_gather(st_v, [rr, col])
                return c

            lax.fori_loop(jnp.int32(0), jnp.int32(8), tpose_body,
                          jnp.int32(0))
            pltpu.async_copy(out_v,
                             out_hbm.at[pl.ds(g * jnp.int32(8), 8), :, w],
                             sem_out)

        fire_in(jnp.int32(0), jnp.int32(0))
        wait_in(jnp.int32(0))
        hash_group(jnp.int32(0), jnp.int32(0))
        fire_gathers(jnp.int32(0))
        fire_in(jnp.int32(1), jnp.int32(1))

        def group_body(g, carry):
            P = g & jnp.int32(1)
            Q = jnp.int32(1) - P

            @pl.when(g < JT - 1)
            def _():
                wait_in(Q)
                hash_group(g + jnp.int32(1), Q)

            drain_gathers()
            stage_rows()

            @pl.when(g < JT - 1)
            def _():
                fire_gathers(Q)

            @pl.when(g + 2 <= JT - 1)
            def _():
                fire_in(g + jnp.int32(2), P)

            @pl.when(g > 0)
            def _():
                drain_out()

            transpose_and_out(g)
            return carry

        lax.fori_loop(jnp.int32(0), jnp.int32(JT), group_body, jnp.int32(0))
        drain_out()

    return sc_gather


_SC_GATHER = _make_sc_call()


def kernel(input_ids, table):
    ids32 = input_ids.astype(jnp.int32)
    ids_x = ids32.reshape(IT, 128, JT, 8).transpose(2, 0, 3, 1)
    out5 = _SC_GATHER(ids_x, table)
    return out5.transpose(2, 4, 0, 1, 3).reshape(NROWS, ROW, EMBED_DIM)
